# baseline (speedup 1.0000x reference)
import jax
import jax.numpy as jnp
from jax.experimental import pallas as pl
from jax.experimental.pallas import tpu as pltpu

BM = 200
NBUF = 4


def _gcn_kernel(x_hbm, adj_hbm, w_ref, b_ref, out_ref,
                x_vmem, adj_buf, support_ref, sems, x_sem):
    n = x_hbm.shape[0]
    nsteps = n // BM

    for s in range(NBUF):
        pltpu.make_async_copy(
            adj_hbm.at[pl.ds(s * BM, BM), :], adj_buf.at[s], sems.at[s]
        ).start()
    pltpu.make_async_copy(x_hbm, x_vmem, x_sem).start()
    pltpu.make_async_copy(x_hbm, x_vmem, x_sem).wait()

    support_ref[...] = jnp.dot(
        x_vmem[...], w_ref[...], preferred_element_type=jnp.float32
    ).astype(jnp.bfloat16)

    def step(i, carry):
        s = jax.lax.rem(i, NBUF)
        pltpu.make_async_copy(
            adj_hbm.at[pl.ds(i * BM, BM), :], adj_buf.at[s], sems.at[s]
        ).wait()
        acc = jnp.dot(
            adj_buf[s].astype(jnp.bfloat16),
            support_ref[...],
            preferred_element_type=jnp.float32,
        )
        out_ref[pl.ds(i * BM, BM), :] = jnp.maximum(acc + b_ref[...], 0.0)

        @pl.when(i + NBUF < nsteps)
        def _():
            pltpu.make_async_copy(
                adj_hbm.at[pl.ds((i + NBUF) * BM, BM), :],
                adj_buf.at[s],
                sems.at[s],
            ).start()

        return carry

    jax.lax.fori_loop(0, nsteps, step, 0)


@jax.jit
def kernel(x, adj, W, b):
    n, nfeat = x.shape
    nhid = W.shape[1]
    b2 = b.reshape(1, nhid)
    return pl.pallas_call(
        _gcn_kernel,
        in_specs=[
            pl.BlockSpec(memory_space=pltpu.HBM),
            pl.BlockSpec(memory_space=pltpu.HBM),
            pl.BlockSpec(memory_space=pltpu.VMEM),
            pl.BlockSpec(memory_space=pltpu.VMEM),
        ],
        out_specs=pl.BlockSpec(memory_space=pltpu.VMEM),
        out_shape=jax.ShapeDtypeStruct((n, nhid), jnp.float32),
        scratch_shapes=[
            pltpu.VMEM((n, nfeat), jnp.float32),
            pltpu.VMEM((NBUF, BM, n), jnp.float32),
            pltpu.VMEM((n, nhid), jnp.bfloat16),
            pltpu.SemaphoreType.DMA((NBUF,)),
            pltpu.SemaphoreType.DMA,
        ],
    )(x, adj, W, b2)


# speedup vs baseline: 1.0001x; 1.0001x over previous
import jax
import jax.numpy as jnp
from jax.experimental import pallas as pl
from jax.experimental.pallas import tpu as pltpu

BM = 200
NBUF = 4


def _gcn_kernel(x_hbm, adj_hbm, w_ref, b_ref, out_ref,
                x_vmem, adj_buf, support_ref, sems, x_sem):
    n = x_hbm.shape[0]
    nsteps = n // BM

    for s in range(NBUF):
        pltpu.make_async_copy(
            adj_hbm.at[pl.ds(s * BM, BM), :], adj_buf.at[s], sems.at[s]
        ).start()
    pltpu.make_async_copy(x_hbm, x_vmem, x_sem).start()
    pltpu.make_async_copy(x_hbm, x_vmem, x_sem).wait()

    support_ref[...] = jnp.dot(
        x_vmem[...], w_ref[...], preferred_element_type=jnp.float32
    )

    def step(i, carry):
        nxt = i + NBUF - 1
        sp = jax.lax.rem(i + NBUF - 1, NBUF)

        @pl.when(jnp.logical_and(i > 0, nxt < nsteps))
        def _():
            pltpu.make_async_copy(
                adj_hbm.at[pl.ds(nxt * BM, BM), :], adj_buf.at[sp],
                sems.at[sp],
            ).start()

        s = jax.lax.rem(i, NBUF)
        pltpu.make_async_copy(
            adj_hbm.at[pl.ds(i * BM, BM), :], adj_buf.at[s], sems.at[s]
        ).wait()
        acc = jnp.dot(
            adj_buf[s], support_ref[...], preferred_element_type=jnp.float32
        )
        out_ref[pl.ds(i * BM, BM), :] = jnp.maximum(acc + b_ref[...], 0.0)
        return carry

    jax.lax.fori_loop(0, nsteps, step, 0)


@jax.jit
def kernel(x, adj, W, b):
    n, nfeat = x.shape
    nhid = W.shape[1]
    b2 = b.reshape(1, nhid)
    return pl.pallas_call(
        _gcn_kernel,
        in_specs=[
            pl.BlockSpec(memory_space=pltpu.HBM),
            pl.BlockSpec(memory_space=pltpu.HBM),
            pl.BlockSpec(memory_space=pltpu.VMEM),
            pl.BlockSpec(memory_space=pltpu.VMEM),
        ],
        out_specs=pl.BlockSpec(memory_space=pltpu.VMEM),
        out_shape=jax.ShapeDtypeStruct((n, nhid), jnp.float32),
        scratch_shapes=[
            pltpu.VMEM((n, nfeat), jnp.float32),
            pltpu.VMEM((NBUF, BM, n), jnp.float32),
            pltpu.VMEM((n, nhid), jnp.float32),
            pltpu.SemaphoreType.DMA((NBUF,)),
            pltpu.SemaphoreType.DMA,
        ],
    )(x, adj, W, b2)


# speedup vs baseline: 1.0247x; 1.0246x over previous
import jax
import jax.numpy as jnp
from jax.experimental import pallas as pl
from jax.experimental.pallas import tpu as pltpu

BM = 200
NBUF = 4


def _gcn_kernel(x_hbm, adj_hbm, w_ref, b_ref, out_ref,
                x_vmem, adj_buf, support_ref, sems, x_sem):
    n = x_hbm.shape[0]
    nsteps = n // BM

    pltpu.make_async_copy(x_hbm, x_vmem, x_sem).start()
    for s in range(NBUF):
        pltpu.make_async_copy(
            adj_hbm.at[pl.ds(s * BM, BM), :], adj_buf.at[s], sems.at[s]
        ).start()
    pltpu.make_async_copy(x_hbm, x_vmem, x_sem).wait()

    support_ref[...] = jnp.dot(
        x_vmem[...], w_ref[...], preferred_element_type=jnp.float32
    )

    def step(i, carry):
        nxt = i + NBUF - 1
        sp = jax.lax.rem(i + NBUF - 1, NBUF)

        @pl.when(jnp.logical_and(i > 0, nxt < nsteps))
        def _():
            pltpu.make_async_copy(
                adj_hbm.at[pl.ds(nxt * BM, BM), :], adj_buf.at[sp],
                sems.at[sp],
            ).start()

        s = jax.lax.rem(i, NBUF)
        pltpu.make_async_copy(
            adj_hbm.at[pl.ds(i * BM, BM), :], adj_buf.at[s], sems.at[s]
        ).wait()
        acc = jnp.dot(
            adj_buf[s], support_ref[...], preferred_element_type=jnp.float32
        )
        out_ref[pl.ds(i * BM, BM), :] = jnp.maximum(acc + b_ref[...], 0.0)
        return carry

    jax.lax.fori_loop(0, nsteps, step, 0)


@jax.jit
def kernel(x, adj, W, b):
    n, nfeat = x.shape
    nhid = W.shape[1]
    b2 = b.reshape(1, nhid)
    return pl.pallas_call(
        _gcn_kernel,
        in_specs=[
            pl.BlockSpec(memory_space=pltpu.HBM),
            pl.BlockSpec(memory_space=pltpu.HBM),
            pl.BlockSpec(memory_space=pltpu.VMEM),
            pl.BlockSpec(memory_space=pltpu.VMEM),
        ],
        out_specs=pl.BlockSpec(memory_space=pltpu.VMEM),
        out_shape=jax.ShapeDtypeStruct((n, nhid), jnp.float32),
        scratch_shapes=[
            pltpu.VMEM((n, nfeat), jnp.float32),
            pltpu.VMEM((NBUF, BM, n), jnp.float32),
            pltpu.VMEM((n, nhid), jnp.float32),
            pltpu.SemaphoreType.DMA((NBUF,)),
            pltpu.SemaphoreType.DMA,
        ],
    )(x, adj, W, b2)


# speedup vs baseline: 1.0453x; 1.0201x over previous
import jax
import jax.numpy as jnp
from jax.experimental import pallas as pl
from jax.experimental.pallas import tpu as pltpu

BM = 400


def _gcn_kernel(x_ref, adj_ref, w_ref, b_ref, out_ref, support_ref):
    @pl.when(pl.program_id(0) == 0)
    def _():
        support_ref[...] = jnp.dot(
            x_ref[...], w_ref[...], preferred_element_type=jnp.float32
        )

    acc = jnp.dot(
        adj_ref[...], support_ref[...], preferred_element_type=jnp.float32
    )
    out_ref[...] = jnp.maximum(acc + b_ref[...], 0.0)


@jax.jit
def kernel(x, adj, W, b):
    n, nfeat = x.shape
    nhid = W.shape[1]
    b2 = b.reshape(1, nhid)
    grid = (n // BM,)
    return pl.pallas_call(
        _gcn_kernel,
        grid=grid,
        in_specs=[
            pl.BlockSpec((n, nfeat), lambda i: (0, 0)),
            pl.BlockSpec((BM, n), lambda i: (i, 0)),
            pl.BlockSpec((nfeat, nhid), lambda i: (0, 0)),
            pl.BlockSpec((1, nhid), lambda i: (0, 0)),
        ],
        out_specs=pl.BlockSpec((BM, nhid), lambda i: (i, 0)),
        out_shape=jax.ShapeDtypeStruct((n, nhid), jnp.float32),
        scratch_shapes=[pltpu.VMEM((n, nhid), jnp.float32)],
        compiler_params=pltpu.CompilerParams(
            dimension_semantics=("arbitrary",),
        ),
    )(x, adj, W, b2)
